# Initial kernel scaffold; baseline (speedup 1.0000x reference)
import jax
import jax.numpy as jnp
from jax.experimental import pallas as pl


def kernel(x, coords, edge_index, w1, gamma, beta, w2, b2):
    raise NotImplementedError("write your pallas kernel here")


# speedup vs baseline: 3.3611x; 3.3611x over previous
import functools

import jax
import jax.numpy as jnp
from jax import lax
from jax.experimental import pallas as pl
from jax.experimental.pallas import tpu as pltpu
from jax.experimental.pallas import tpu_sc as plsc

C = 128
N = 10000
K = 16
NW = 32
NPAD = 10240
NPW = NPAD // NW
CN = 4
NCHUNK = NPW // CN
MROWS = 10256
CPROWS = 10248
SIGMA = 0.02
EPS = 1e-5
NEG_SLOPE = 0.2
TOTAL_EDGES = N * K


def _s1_body(x_ref, w1_ref, u_ref, v_ref):
    w1v = w1_ref[...]
    wb = w1v[:, C:]
    wa = w1v[:, :C] - wb
    xb = x_ref[...]
    dn = (((0,), (1,)), ((), ()))
    u_ref[...] = lax.dot_general(xb, wa, dn, preferred_element_type=jnp.float32)
    v_ref[...] = lax.dot_general(xb, wb, dn, preferred_element_type=jnp.float32)


def _stage1(x2p, w1):
    bn = 1024
    return pl.pallas_call(
        _s1_body,
        grid=(NPAD // bn,),
        in_specs=[
            pl.BlockSpec((C, bn), lambda i: (0, i)),
            pl.BlockSpec((C, 2 * C), lambda i: (0, 0)),
        ],
        out_specs=[
            pl.BlockSpec((bn, C), lambda i: (i, 0)),
            pl.BlockSpec((bn, C), lambda i: (i, 0)),
        ],
        out_shape=[
            jax.ShapeDtypeStruct((NPAD, C), jnp.float32),
            jax.ShapeDtypeStruct((NPAD, C), jnp.float32),
        ],
    )(x2p, w1)


def _sc_edge_body(ut_hbm, vt_hbm, idxi_hbm, idxj_hbm,
                  m_hbm, su_hbm, sq_hbm,
                  idxi_v, idxj_v, ub, vb, mbuf, subuf, sqbuf, sem_u, sem_v):
    wid = lax.axis_index("s") * 2 + lax.axis_index("c")
    e_base = wid * (NPW * K)
    n_base = 8 + wid * NPW

    def chunk(ci, accs):
        eb = e_base + ci * (CN * K)
        nb = n_base + ci * CN
        pltpu.sync_copy(idxi_hbm.at[pl.ds(eb, CN * K)], idxi_v)
        pltpu.sync_copy(idxj_hbm.at[pl.ds(eb, CN * K)], idxj_v)
        cu = pltpu.async_copy(ut_hbm.at[idxi_v], ub, sem_u)
        cv = pltpu.async_copy(vt_hbm.at[idxj_v], vb, sem_v)
        cu.wait()
        cv.wait()
        out = list(accs)
        for r in range(C // 16):
            s = out[r]
            q = out[8 + r]
            sl = pl.ds(r * 16, 16)
            for ni in range(CN):
                mreg = jnp.full((16,), -3.0e38, jnp.float32)
                for k in range(K):
                    e = ni * K + k
                    g = ub[e, sl] + vb[e, sl]
                    mreg = jnp.maximum(mreg, g)
                    s = s + g
                    q = q + g * g
                mbuf[ni, sl] = mreg
            out[r] = s
            out[8 + r] = q
        pltpu.sync_copy(mbuf, m_hbm.at[pl.ds(nb, CN)])
        return tuple(out)

    zero = jnp.zeros((16,), jnp.float32)
    accs = lax.fori_loop(0, NCHUNK, chunk, (zero,) * 16)
    for r in range(C // 16):
        sl = pl.ds(r * 16, 16)
        subuf[0, sl] = accs[r]
        sqbuf[0, sl] = accs[8 + r]
    pltpu.sync_copy(subuf, su_hbm.at[pl.ds(wid, 1)])
    pltpu.sync_copy(sqbuf, sq_hbm.at[pl.ds(wid, 1)])


def _stage2(ut, vt, idx_i, idx_j):
    mesh = plsc.VectorSubcoreMesh(core_axis_name="c", subcore_axis_name="s")
    fn = pl.kernel(
        _sc_edge_body,
        out_type=[
            jax.ShapeDtypeStruct((MROWS, C), jnp.float32),
            jax.ShapeDtypeStruct((NW, C), jnp.float32),
            jax.ShapeDtypeStruct((NW, C), jnp.float32),
        ],
        mesh=mesh,
        scratch_types=[
            pltpu.VMEM((CN * K,), jnp.int32),
            pltpu.VMEM((CN * K,), jnp.int32),
            pltpu.VMEM((CN * K, C), jnp.float32),
            pltpu.VMEM((CN * K, C), jnp.float32),
            pltpu.VMEM((CN, C), jnp.float32),
            pltpu.VMEM((1, C), jnp.float32),
            pltpu.VMEM((1, C), jnp.float32),
            pltpu.SemaphoreType.DMA,
            pltpu.SemaphoreType.DMA,
        ],
    )
    return fn(ut, vt, idx_i, idx_j)


def _s3_body(m_ref, su_ref, sq_ref, cp_ref, w2_ref, b2_ref, gam_ref, bet_ref, o_ref):
    i = pl.program_id(0)
    bn = 1024
    inv_cnt = 1.0 / TOTAL_EDGES
    su = jnp.sum(su_ref[...], axis=0, keepdims=True)
    sq = jnp.sum(sq_ref[...], axis=0, keepdims=True)
    mean = su * inv_cnt
    var = sq * inv_cnt - mean * mean
    scale = gam_ref[...] / jnp.sqrt(var + EPS)
    shift = bet_ref[...] - mean * scale

    cc = cp_ref[pl.ds(i * bn + 2, bn), :]
    acc = jnp.zeros((C, bn), jnp.float32)
    for j in range(5):
        ms = m_ref[pl.ds(i * bn + 6 + j, bn), :]
        rows = i * bn + 6 + j + lax.broadcasted_iota(jnp.int32, (bn, 1), 0)
        valid = (rows >= 8) & (rows < N + 8)
        h = ms * scale + shift
        h = jnp.where(h >= 0, h, NEG_SLOPE * h)
        h = jnp.where(valid, h, 0.0)
        cj = cp_ref[pl.ds(i * bn + j, bn), :]
        dist = jnp.sum((cj - cc) ** 2, axis=1, keepdims=True)
        w = jnp.exp(dist * (-1.0 / (SIGMA * SIGMA)))
        acc = acc + lax.dot_general(w2_ref[j], h * w, (((1,), (1,)), ((), ())),
                                    preferred_element_type=jnp.float32)
    o_ref[...] = acc + b2_ref[...]


def _stage3(m, su, sq, cpad, w2t, b2c, gam, bet):
    bn = 1024
    full = lambda shape: pl.BlockSpec(shape, lambda i: tuple(0 for _ in shape))
    return pl.pallas_call(
        _s3_body,
        grid=(NPAD // bn,),
        in_specs=[
            full((MROWS, C)),
            full((NW, C)),
            full((NW, C)),
            full((CPROWS, 8)),
            full((5, C, C)),
            full((C, 1)),
            full((1, C)),
            full((1, C)),
        ],
        out_specs=pl.BlockSpec((C, bn), lambda i: (0, i)),
        out_shape=jax.ShapeDtypeStruct((C, NPAD), jnp.float32),
    )(m, su, sq, cpad, w2t, b2c, gam, bet)


@jax.jit
def kernel(x, coords, edge_index, w1, gamma, beta, w2, b2):
    x2p = jnp.pad(x[0], ((0, 0), (0, NPAD - N)))
    idx_i = jnp.pad(edge_index[1, 0].reshape(-1), (0, (NPAD - N) * K),
                    constant_values=N)
    idx_j = jnp.pad(edge_index[0, 0].reshape(-1), (0, (NPAD - N) * K),
                    constant_values=N)
    ut, vt = _stage1(x2p, w1)
    m, su, sq = _stage2(ut, vt, idx_i, idx_j)
    cpad = jnp.pad(coords[0].T, ((2, CPROWS - N - 2), (0, 5)))
    w2t = jnp.transpose(w2, (2, 0, 1))
    out = _stage3(m, su, sq, cpad, w2t, b2[:, None], gamma[None, :], beta[None, :])
    return out[:, :N][None]


# speedup vs baseline: 7.6348x; 2.2715x over previous
import functools

import jax
import jax.numpy as jnp
from jax import lax
from jax.experimental import pallas as pl
from jax.experimental.pallas import tpu as pltpu
from jax.experimental.pallas import tpu_sc as plsc

C = 128
N = 10000
K = 16
NW = 32
NPAD = 10240
NPW = NPAD // NW
CN = 4
NCHUNK = NPW // CN
MROWS = 10256
CPROWS = 10248
SIGMA = 0.02
EPS = 1e-5
NEG_SLOPE = 0.2
TOTAL_EDGES = N * K


def _s1_body(x_ref, w1_ref, u_ref, v_ref):
    w1v = w1_ref[...]
    wb = w1v[:, C:]
    wa = w1v[:, :C] - wb
    xb = x_ref[...]
    dn = (((0,), (1,)), ((), ()))
    u_ref[...] = lax.dot_general(xb, wa, dn, preferred_element_type=jnp.float32)
    v_ref[...] = lax.dot_general(xb, wb, dn, preferred_element_type=jnp.float32)


def _stage1(x2p, w1):
    bn = 1024
    return pl.pallas_call(
        _s1_body,
        grid=(NPAD // bn,),
        in_specs=[
            pl.BlockSpec((C, bn), lambda i: (0, i)),
            pl.BlockSpec((C, 2 * C), lambda i: (0, 0)),
        ],
        out_specs=[
            pl.BlockSpec((bn, C), lambda i: (i, 0)),
            pl.BlockSpec((bn, C), lambda i: (i, 0)),
        ],
        out_shape=[
            jax.ShapeDtypeStruct((NPAD, C), jnp.float32),
            jax.ShapeDtypeStruct((NPAD, C), jnp.float32),
        ],
    )(x2p, w1)


def _sc_edge_body(ut_hbm, vt_hbm, idxi_hbm, idxj_hbm,
                  m_hbm, su_hbm, sq_hbm,
                  idxi_all, idxj_all, ub0, vb0, ub1, vb1, mb0, mb1,
                  subuf, sqbuf, sem_g0, sem_g1, sem_m0, sem_m1):
    wid = lax.axis_index("s") * 2 + lax.axis_index("c")
    e_base = wid * (NPW * K)
    n_base = 8 + wid * NPW

    pltpu.sync_copy(idxi_hbm.at[pl.ds(e_base, NPW * K)], idxi_all)
    pltpu.sync_copy(idxj_hbm.at[pl.ds(e_base, NPW * K)], idxj_all)

    def issue(ci, ub, vb, sem):
        sl = pl.ds(ci * (CN * K), CN * K)
        pltpu.async_copy(ut_hbm.at[idxi_all.at[sl]], ub, sem)
        pltpu.async_copy(vt_hbm.at[idxj_all.at[sl]], vb, sem)

    def wait_gather(ci, ub, vb, sem):
        sl = pl.ds(ci * (CN * K), CN * K)
        pltpu.make_async_copy(ut_hbm.at[idxi_all.at[sl]], ub, sem).wait()
        pltpu.make_async_copy(vt_hbm.at[idxj_all.at[sl]], vb, sem).wait()

    def compute(ub, vb, mb, accs):
        accs = list(accs)

        def nbody(ni, carry):
            carry = list(carry)
            mregs = [jnp.full((16,), -3.0e38, jnp.float32) for _ in range(8)]
            for k in range(K):
                e = ni * K + k
                for r in range(8):
                    sl = pl.ds(r * 16, 16)
                    g = ub[e, sl] + vb[e, sl]
                    mregs[r] = jnp.maximum(mregs[r], g)
                    carry[r] = carry[r] + g
                    carry[8 + r] = carry[8 + r] + g * g
            for r in range(8):
                mb[ni, pl.ds(r * 16, 16)] = mregs[r]
            return tuple(carry)

        return lax.fori_loop(0, CN, nbody, tuple(accs))

    bufs = ((ub0, vb0, mb0, sem_g0, sem_m0),
            (ub1, vb1, mb1, sem_g1, sem_m1))

    issue(0, ub0, vb0, sem_g0)
    issue(1, ub1, vb1, sem_g1)

    def outer(t, accs):
        for b, (ub, vb, mb, sem_g, sem_m) in enumerate(bufs):
            ci = 2 * t + b
            wait_gather(ci, ub, vb, sem_g)

            @pl.when(t > 0)
            def _():
                pltpu.make_async_copy(mb, m_hbm.at[pl.ds(n_base, CN)],
                                      sem_m).wait()

            accs = compute(ub, vb, mb, accs)

            @pl.when(ci + 2 < NCHUNK)
            def _():
                issue(ci + 2, ub, vb, sem_g)

            pltpu.async_copy(mb, m_hbm.at[pl.ds(n_base + ci * CN, CN)], sem_m)
        return accs

    zero = jnp.zeros((16,), jnp.float32)
    accs = lax.fori_loop(0, NCHUNK // 2, outer, (zero,) * 16)
    pltpu.make_async_copy(mb0, m_hbm.at[pl.ds(n_base, CN)], sem_m0).wait()
    pltpu.make_async_copy(mb1, m_hbm.at[pl.ds(n_base, CN)], sem_m1).wait()
    for r in range(C // 16):
        sl = pl.ds(r * 16, 16)
        subuf[0, sl] = accs[r]
        sqbuf[0, sl] = accs[8 + r]
    pltpu.sync_copy(subuf, su_hbm.at[pl.ds(wid, 1)])
    pltpu.sync_copy(sqbuf, sq_hbm.at[pl.ds(wid, 1)])


def _stage2(ut, vt, idx_i, idx_j):
    mesh = plsc.VectorSubcoreMesh(core_axis_name="c", subcore_axis_name="s")
    fn = pl.kernel(
        _sc_edge_body,
        out_type=[
            jax.ShapeDtypeStruct((MROWS, C), jnp.float32),
            jax.ShapeDtypeStruct((NW, C), jnp.float32),
            jax.ShapeDtypeStruct((NW, C), jnp.float32),
        ],
        mesh=mesh,
        scratch_types=[
            pltpu.VMEM((NPW * K,), jnp.int32),
            pltpu.VMEM((NPW * K,), jnp.int32),
            pltpu.VMEM((CN * K, C), jnp.float32),
            pltpu.VMEM((CN * K, C), jnp.float32),
            pltpu.VMEM((CN * K, C), jnp.float32),
            pltpu.VMEM((CN * K, C), jnp.float32),
            pltpu.VMEM((CN, C), jnp.float32),
            pltpu.VMEM((CN, C), jnp.float32),
            pltpu.VMEM((1, C), jnp.float32),
            pltpu.VMEM((1, C), jnp.float32),
            pltpu.SemaphoreType.DMA,
            pltpu.SemaphoreType.DMA,
            pltpu.SemaphoreType.DMA,
            pltpu.SemaphoreType.DMA,
        ],
    )
    return fn(ut, vt, idx_i, idx_j)


def _s3_body(m_ref, su_ref, sq_ref, cp_ref, w2_ref, b2_ref, gam_ref, bet_ref, o_ref):
    i = pl.program_id(0)
    bn = 1024
    inv_cnt = 1.0 / TOTAL_EDGES
    su = jnp.sum(su_ref[...], axis=0, keepdims=True)
    sq = jnp.sum(sq_ref[...], axis=0, keepdims=True)
    mean = su * inv_cnt
    var = sq * inv_cnt - mean * mean
    scale = gam_ref[...] / jnp.sqrt(var + EPS)
    shift = bet_ref[...] - mean * scale

    cc = cp_ref[pl.ds(i * bn + 2, bn), :]
    acc = jnp.zeros((C, bn), jnp.float32)
    for j in range(5):
        ms = m_ref[pl.ds(i * bn + 6 + j, bn), :]
        rows = i * bn + 6 + j + lax.broadcasted_iota(jnp.int32, (bn, 1), 0)
        valid = (rows >= 8) & (rows < N + 8)
        h = ms * scale + shift
        h = jnp.where(h >= 0, h, NEG_SLOPE * h)
        h = jnp.where(valid, h, 0.0)
        cj = cp_ref[pl.ds(i * bn + j, bn), :]
        dist = jnp.sum((cj - cc) ** 2, axis=1, keepdims=True)
        w = jnp.exp(dist * (-1.0 / (SIGMA * SIGMA)))
        acc = acc + lax.dot_general(w2_ref[j], h * w, (((1,), (1,)), ((), ())),
                                    preferred_element_type=jnp.float32)
    o_ref[...] = acc + b2_ref[...]


def _stage3(m, su, sq, cpad, w2t, b2c, gam, bet):
    bn = 1024
    full = lambda shape: pl.BlockSpec(shape, lambda i: tuple(0 for _ in shape))
    return pl.pallas_call(
        _s3_body,
        grid=(NPAD // bn,),
        in_specs=[
            full((MROWS, C)),
            full((NW, C)),
            full((NW, C)),
            full((CPROWS, 8)),
            full((5, C, C)),
            full((C, 1)),
            full((1, C)),
            full((1, C)),
        ],
        out_specs=pl.BlockSpec((C, bn), lambda i: (0, i)),
        out_shape=jax.ShapeDtypeStruct((C, NPAD), jnp.float32),
    )(m, su, sq, cpad, w2t, b2c, gam, bet)


@jax.jit
def kernel(x, coords, edge_index, w1, gamma, beta, w2, b2):
    x2p = jnp.pad(x[0], ((0, 0), (0, NPAD - N)))
    idx_i = jnp.pad(edge_index[1, 0].reshape(-1), (0, (NPAD - N) * K),
                    constant_values=N)
    idx_j = jnp.pad(edge_index[0, 0].reshape(-1), (0, (NPAD - N) * K),
                    constant_values=N)
    ut, vt = _stage1(x2p, w1)
    m, su, sq = _stage2(ut, vt, idx_i, idx_j)
    cpad = jnp.pad(coords[0].T, ((2, CPROWS - N - 2), (0, 5)))
    w2t = jnp.transpose(w2, (2, 0, 1))
    out = _stage3(m, su, sq, cpad, w2t, b2[:, None], gamma[None, :], beta[None, :])
    return out[:, :N][None]
